# baseline (speedup 1.0000x reference)
import functools
import math

import jax
import jax.numpy as jnp
from jax import lax
from jax.experimental import pallas as pl
from jax.experimental.pallas import tpu as pltpu
from jax.experimental.pallas import tpu_sc as plsc

S = 64.0
MARGIN = 0.5
COS_M = math.cos(MARGIN)
SIN_M = math.sin(MARGIN)

NUM_WORKERS = 32
LANES = 16
BUF = 20000


def _sqrt_sc(a):
    seed = plsc.bitcast(
        0x5F3759DF - lax.shift_right_logical(plsc.bitcast(a, jnp.int32), 1),
        jnp.float32,
    )
    z = seed
    for _ in range(4):
        z = z * (1.5 - 0.5 * a * z * z)
    return a * z


def _arcface_sc(logits_flat, labels, n_rows, n_cols):
    rows_per_w = n_rows // NUM_WORKERS
    chunk = rows_per_w * n_cols
    n_elems = n_rows * n_cols
    mesh = plsc.VectorSubcoreMesh(core_axis_name="c", subcore_axis_name="s")

    @functools.partial(
        pl.kernel,
        out_type=jax.ShapeDtypeStruct((n_elems,), jnp.float32),
        mesh=mesh,
        scratch_types=[
            pltpu.VMEM((rows_per_w,), jnp.int32),
            pltpu.VMEM((rows_per_w,), jnp.int32),
            pltpu.VMEM((rows_per_w,), jnp.float32),
            pltpu.VMEM((rows_per_w,), jnp.float32),
            pltpu.VMEM((BUF,), jnp.float32),
            pltpu.SemaphoreType.DMA,
        ],
        compiler_params=pltpu.CompilerParams(needs_layout_passes=False),
    )
    def k(logits_hbm, labels_hbm, out_hbm, lab_v, gidx_v, x_v, pvs_v, buf, sem):
        wid = lax.axis_index("s") * 2 + lax.axis_index("c")
        base_row = wid * rows_per_w
        base = base_row * n_cols

        pltpu.sync_copy(labels_hbm.at[pl.ds(base_row, rows_per_w)], lab_v)
        for i in range(rows_per_w // LANES):
            lab = lab_v[pl.ds(i * LANES, LANES)]
            rows = base_row + i * LANES + lax.iota(jnp.int32, LANES)
            gidx_v[pl.ds(i * LANES, LANES)] = jnp.where(
                lab >= 0, rows * n_cols + lab, -1
            )
        pltpu.async_copy(logits_hbm.at[gidx_v], x_v, sem).wait()
        for i in range(rows_per_w // LANES):
            x = x_v[pl.ds(i * LANES, LANES)]
            a = jnp.maximum(1.0 - x * x, 1e-12)
            pvs_v[pl.ds(i * LANES, LANES)] = (
                x * (S * COS_M) - _sqrt_sc(a) * (S * SIN_M)
            )

        @pl.loop(0, chunk, step=BUF)
        def _tile(off):
            start = base + off
            pltpu.sync_copy(logits_hbm.at[pl.ds(start, BUF)], buf)

            @pl.loop(0, BUF, step=LANES, unroll=8)
            def _scale(v):
                buf[pl.ds(v, LANES)] = buf[pl.ds(v, LANES)] * S

            for i in range(rows_per_w // LANES):
                gi = gidx_v[pl.ds(i * LANES, LANES)]
                m = (gi >= start) & (gi < start + BUF)
                plsc.store_scatter(
                    buf, [gi - start], pvs_v[pl.ds(i * LANES, LANES)], mask=m
                )
            pltpu.sync_copy(buf, out_hbm.at[pl.ds(start, BUF)])

    return k(logits_flat, labels)


@jax.jit
def kernel(logits, labels):
    n_rows, n_cols = logits.shape
    out = _arcface_sc(logits.reshape(-1), labels, n_rows, n_cols)
    return out.reshape(n_rows, n_cols)
